# baseline (speedup 1.0000x reference)
import jax
import jax.numpy as jnp
from jax import lax
from jax.experimental import pallas as pl
from jax.experimental.pallas import tpu as pltpu
from jax.experimental.pallas import tpu_sc as plsc

D = 32
_NC = 2
_NS = 16
_NW = _NC * _NS
_RPC = 4
_IPC = _RPC * 26
_CPW = 128

_V = 1000000
_TR_FULL = _V // 128
_TAIL = _V - _TR_FULL * 128


def _transpose_chunk(in_ref, out_ref):
    ii = lax.iota(jnp.int32, 16)
    for vrow in range(32):
        for h in range(8):
            c_idx = ii + 16 * (h % 2)
            i_idx = jnp.full((16,), 4 * vrow + (h // 2), jnp.int32)
            val = plsc.load_gather(in_ref, [c_idx, i_idx])
            out_ref[vrow, pl.ds(16 * h, 16)] = val


def _relayout_body(tabt_hbm, tail_hbm, out_hbm, in_v, out_v, si0, si1, so0, so1):
    wid = lax.axis_index("s") * _NC + lax.axis_index("c")
    sems_in = (si0, si1)
    sems_out = (so0, so1)
    extra = jnp.where(wid < 4, 1, 0)
    n_chunks = 244 + extra
    base = wid * 244 + jnp.minimum(wid, 4)

    def start_in(k, b):
        tr = base + k
        pltpu.async_copy(
            tabt_hbm.at[:, pl.ds(pl.multiple_of(tr * 128, 128), 128)],
            in_v.at[b],
            sems_in[b],
        )

    def wait_in(b):
        pltpu.make_async_copy(
            tabt_hbm.at[:, pl.ds(0, 128)], in_v.at[b], sems_in[b]
        ).wait()

    def start_out(k, b):
        tr = base + k
        pltpu.async_copy(
            out_v.at[b],
            out_hbm.at[pl.ds(pl.multiple_of(tr * 32, 32), 32)],
            sems_out[b],
        )

    def drain_out(b):
        pltpu.make_async_copy(
            out_v.at[b], out_hbm.at[pl.ds(0, 32)], sems_out[b]
        ).wait()

    start_in(0, 0)

    def step(g, carry):
        for b in range(2):
            k = 2 * g + b

            @pl.when(k < n_chunks)
            def _():
                wait_in(b)

                @pl.when(k + 1 < n_chunks)
                def _():
                    start_in(k + 1, 1 - b)

                @pl.when(k >= 2)
                def _():
                    drain_out(b)

                _transpose_chunk(in_v.at[b], out_v.at[b])
                start_out(k, b)

        return carry

    lax.fori_loop(0, 123, step, 0)
    drain_out(0)
    drain_out(1)

    @pl.when(wid == 0)
    def _():
        pltpu.sync_copy(tail_hbm, out_hbm.at[pl.ds(_TR_FULL * 32, 16)])


def _gather_body(table_hbm, idx_hbm, out_hbm, idx_v, rows_v,
                 sem_in0, sem_in1, sem_out0, sem_out1):
    wid = lax.axis_index("s") * _NC + lax.axis_index("c")
    row0 = wid * (_CPW * _RPC)
    sems_in = (sem_in0, sem_in1)
    sems_out = (sem_out0, sem_out1)
    pltpu.sync_copy(idx_hbm.at[wid], idx_v)

    def start_gather(k, b):
        pltpu.async_copy(table_hbm.at[idx_v.at[k]], rows_v.at[b], sems_in[b])

    def wait_gather(b):
        pltpu.make_async_copy(
            table_hbm.at[pl.ds(0, _IPC)], rows_v.at[b], sems_in[b]
        ).wait()

    def start_writes(k, b):
        for m in range(_RPC):
            pltpu.async_copy(
                rows_v.at[b].at[pl.ds(26 * m, 26)],
                out_hbm.at[row0 + k * _RPC + m],
                sems_out[b],
            )

    def drain_writes(b):
        for m in range(_RPC):
            pltpu.make_async_copy(
                rows_v.at[b].at[pl.ds(26 * m, 26)], out_hbm.at[0], sems_out[b]
            ).wait()

    start_gather(0, 0)

    def step(g, carry):
        for b in range(2):
            k = 2 * g + b
            wait_gather(b)
            start_writes(k, b)

            @pl.when(k + 1 < _CPW)
            def _():
                nb = 1 - b

                @pl.when(k >= 1)
                def _():
                    drain_writes(nb)

                start_gather(k + 1, nb)

        return carry

    lax.fori_loop(0, _CPW // 2, step, 0)
    drain_writes(0)
    drain_writes(1)


def kernel(x, table):
    rows, cols = x.shape
    idx = x.astype(jnp.int32).reshape(_NW, _CPW, _IPC)
    mesh = plsc.VectorSubcoreMesh(core_axis_name="c", subcore_axis_name="s")

    relayout = pl.kernel(
        _relayout_body,
        mesh=mesh,
        out_type=jax.ShapeDtypeStruct((_V // 4, 128), jnp.float32),
        scratch_types=[
            pltpu.VMEM((2, 32, 128), jnp.float32),
            pltpu.VMEM((2, 32, 128), jnp.float32),
            pltpu.SemaphoreType.DMA,
            pltpu.SemaphoreType.DMA,
            pltpu.SemaphoreType.DMA,
            pltpu.SemaphoreType.DMA,
        ],
        compiler_params=pltpu.CompilerParams(needs_layout_passes=False),
    )

    gather = pl.kernel(
        _gather_body,
        mesh=mesh,
        out_type=jax.ShapeDtypeStruct((rows, cols, D), jnp.float32),
        scratch_types=[
            pltpu.VMEM((_CPW, _IPC), jnp.int32),
            pltpu.VMEM((2, _IPC, D), jnp.float32),
            pltpu.SemaphoreType.DMA,
            pltpu.SemaphoreType.DMA,
            pltpu.SemaphoreType.DMA,
            pltpu.SemaphoreType.DMA,
        ],
        compiler_params=pltpu.CompilerParams(use_tc_tiling_on_sc=False),
    )

    tail = table[_TR_FULL * 128:].reshape(16, 128)
    table_lin = relayout(table.T, tail).reshape(_V, D)
    return gather(table_lin, idx)


# speedup vs baseline: 1.9039x; 1.9039x over previous
import jax
import jax.numpy as jnp
from jax import lax
from jax.experimental import pallas as pl
from jax.experimental.pallas import tpu as pltpu
from jax.experimental.pallas import tpu_sc as plsc

D = 32
_NC = 2
_NS = 16
_NW = _NC * _NS
_RPC = 4
_IPC = _RPC * 26
_CPW = 128

_V = 1000000
_TR_FULL = _V // 128
_TAIL = _V - _TR_FULL * 128


def _transpose_chunk(in_ref, out_ref):
    ii = lax.iota(jnp.int32, 16)

    @plsc.parallel_loop(0, 32, 1, unroll=4)
    def _(vrow):
        for h in range(8):
            c_idx = ii + 16 * (h % 2)
            i_idx = jnp.zeros((16,), jnp.int32) + (4 * vrow + h // 2)
            val = plsc.load_gather(in_ref, [c_idx, i_idx])
            out_ref[vrow, pl.ds(16 * h, 16)] = val


def _relayout_body(tabt_hbm, tail_hbm, out_hbm, in_v, out_v, si0, si1, so0, so1):
    wid = lax.axis_index("s") * _NC + lax.axis_index("c")
    sems_in = (si0, si1)
    sems_out = (so0, so1)
    extra = jnp.where(wid < 4, 1, 0)
    n_chunks = 244 + extra
    base = wid * 244 + jnp.minimum(wid, 4)

    def start_in(k, b):
        tr = base + k
        pltpu.async_copy(
            tabt_hbm.at[:, pl.ds(pl.multiple_of(tr * 128, 128), 128)],
            in_v.at[b],
            sems_in[b],
        )

    def wait_in(b):
        pltpu.make_async_copy(
            tabt_hbm.at[:, pl.ds(0, 128)], in_v.at[b], sems_in[b]
        ).wait()

    def start_out(k, b):
        tr = base + k
        pltpu.async_copy(
            out_v.at[b],
            out_hbm.at[pl.ds(pl.multiple_of(tr * 32, 32), 32)],
            sems_out[b],
        )

    def drain_out(b):
        pltpu.make_async_copy(
            out_v.at[b], out_hbm.at[pl.ds(0, 32)], sems_out[b]
        ).wait()

    start_in(0, 0)

    def step(g, carry):
        for b in range(2):
            k = 2 * g + b

            @pl.when(k < n_chunks)
            def _():
                wait_in(b)

                @pl.when(k + 1 < n_chunks)
                def _():
                    start_in(k + 1, 1 - b)

                @pl.when(k >= 2)
                def _():
                    drain_out(b)

                _transpose_chunk(in_v.at[b], out_v.at[b])
                start_out(k, b)

        return carry

    lax.fori_loop(0, 123, step, 0)
    drain_out(0)
    drain_out(1)

    @pl.when(wid == 0)
    def _():
        pltpu.sync_copy(tail_hbm, out_hbm.at[pl.ds(_TR_FULL * 32, 16)])


def _gather_body(table_hbm, idx_hbm, out_hbm, idx_v, rows_v,
                 sem_in0, sem_in1, sem_out0, sem_out1):
    wid = lax.axis_index("s") * _NC + lax.axis_index("c")
    row0 = wid * (_CPW * _RPC)
    sems_in = (sem_in0, sem_in1)
    sems_out = (sem_out0, sem_out1)
    pltpu.sync_copy(idx_hbm.at[wid], idx_v)

    def start_gather(k, b):
        pltpu.async_copy(table_hbm.at[idx_v.at[k]], rows_v.at[b], sems_in[b])

    def wait_gather(b):
        pltpu.make_async_copy(
            table_hbm.at[pl.ds(0, _IPC)], rows_v.at[b], sems_in[b]
        ).wait()

    def start_writes(k, b):
        for m in range(_RPC):
            pltpu.async_copy(
                rows_v.at[b].at[pl.ds(26 * m, 26)],
                out_hbm.at[row0 + k * _RPC + m],
                sems_out[b],
            )

    def drain_writes(b):
        for m in range(_RPC):
            pltpu.make_async_copy(
                rows_v.at[b].at[pl.ds(26 * m, 26)], out_hbm.at[0], sems_out[b]
            ).wait()

    start_gather(0, 0)

    def step(g, carry):
        for b in range(2):
            k = 2 * g + b
            wait_gather(b)
            start_writes(k, b)

            @pl.when(k + 1 < _CPW)
            def _():
                nb = 1 - b

                @pl.when(k >= 1)
                def _():
                    drain_writes(nb)

                start_gather(k + 1, nb)

        return carry

    lax.fori_loop(0, _CPW // 2, step, 0)
    drain_writes(0)
    drain_writes(1)


def kernel(x, table):
    rows, cols = x.shape
    idx = x.astype(jnp.int32).reshape(_NW, _CPW, _IPC)
    mesh = plsc.VectorSubcoreMesh(core_axis_name="c", subcore_axis_name="s")

    relayout = pl.kernel(
        _relayout_body,
        mesh=mesh,
        out_type=jax.ShapeDtypeStruct((_V // 4, 128), jnp.float32),
        scratch_types=[
            pltpu.VMEM((2, 32, 128), jnp.float32),
            pltpu.VMEM((2, 32, 128), jnp.float32),
            pltpu.SemaphoreType.DMA,
            pltpu.SemaphoreType.DMA,
            pltpu.SemaphoreType.DMA,
            pltpu.SemaphoreType.DMA,
        ],
        compiler_params=pltpu.CompilerParams(needs_layout_passes=False),
    )

    gather = pl.kernel(
        _gather_body,
        mesh=mesh,
        out_type=jax.ShapeDtypeStruct((rows, cols, D), jnp.float32),
        scratch_types=[
            pltpu.VMEM((_CPW, _IPC), jnp.int32),
            pltpu.VMEM((2, _IPC, D), jnp.float32),
            pltpu.SemaphoreType.DMA,
            pltpu.SemaphoreType.DMA,
            pltpu.SemaphoreType.DMA,
            pltpu.SemaphoreType.DMA,
        ],
        compiler_params=pltpu.CompilerParams(use_tc_tiling_on_sc=False),
    )

    tail = table[_TR_FULL * 128:].reshape(16, 128)
    table_lin = relayout(table.T, tail).reshape(_V, D)
    return gather(table_lin, idx)


# speedup vs baseline: 1.9114x; 1.0039x over previous
import jax
import jax.numpy as jnp
from jax import lax
from jax.experimental import pallas as pl
from jax.experimental.pallas import tpu as pltpu
from jax.experimental.pallas import tpu_sc as plsc

D = 32
_NC = 2
_NS = 16
_NW = _NC * _NS
_JC = 26

_V = 1000000
_TR_FULL = _V // 128
_TAIL = _V - _TR_FULL * 128


def _transpose_chunk(in_ref, out_ref):
    ii = lax.iota(jnp.int32, 16)

    @plsc.parallel_loop(0, 32, 1, unroll=4)
    def _(vrow):
        for h in range(8):
            c_idx = ii + 16 * (h % 2)
            i_idx = jnp.zeros((16,), jnp.int32) + (4 * vrow + h // 2)
            val = plsc.load_gather(in_ref, [c_idx, i_idx])
            out_ref[vrow, pl.ds(16 * h, 16)] = val


def _relayout_body(tabt_hbm, tail_hbm, out_hbm, in_v, out_v, si0, si1, so0, so1):
    wid = lax.axis_index("s") * _NC + lax.axis_index("c")
    sems_in = (si0, si1)
    sems_out = (so0, so1)
    extra = jnp.where(wid < 4, 1, 0)
    n_chunks = 244 + extra
    base = wid * 244 + jnp.minimum(wid, 4)

    def start_in(k, b):
        tr = base + k
        pltpu.async_copy(
            tabt_hbm.at[:, pl.ds(pl.multiple_of(tr * 128, 128), 128)],
            in_v.at[b],
            sems_in[b],
        )

    def wait_in(b):
        pltpu.make_async_copy(
            tabt_hbm.at[:, pl.ds(0, 128)], in_v.at[b], sems_in[b]
        ).wait()

    def start_out(k, b):
        tr = base + k
        pltpu.async_copy(
            out_v.at[b],
            out_hbm.at[pl.ds(pl.multiple_of(tr * 32, 32), 32)],
            sems_out[b],
        )

    def drain_out(b):
        pltpu.make_async_copy(
            out_v.at[b], out_hbm.at[pl.ds(0, 32)], sems_out[b]
        ).wait()

    start_in(0, 0)

    def step(g, carry):
        for b in range(2):
            k = 2 * g + b

            @pl.when(k < n_chunks)
            def _():
                wait_in(b)

                @pl.when(k + 1 < n_chunks)
                def _():
                    start_in(k + 1, 1 - b)

                @pl.when(k >= 2)
                def _():
                    drain_out(b)

                _transpose_chunk(in_v.at[b], out_v.at[b])
                start_out(k, b)

        return carry

    lax.fori_loop(0, 123, step, 0)
    drain_out(0)
    drain_out(1)

    @pl.when(wid == 0)
    def _():
        pltpu.sync_copy(tail_hbm, out_hbm.at[pl.ds(_TR_FULL * 32, 16)])


def _transpose_rows(rows_ref, tout_ref):
    ii = lax.iota(jnp.int32, 16)

    @plsc.parallel_loop(0, 32, 1, unroll=4)
    def _(w):
        tc = w // 8
        c8 = lax.rem(w, 8)
        c_idx = jnp.zeros((16,), jnp.int32) + w
        for s in range(8):
            i_idx = ii + 16 * s
            val = plsc.load_gather(rows_ref, [i_idx, c_idx])
            tout_ref[tc, c8, pl.ds(16 * s, 16)] = val


def _gather_body(table_hbm, idx_hbm, out_hbm, idx_v, rows_v, tout_v,
                 si0, si1, so0, so1):
    wid = lax.axis_index("s") * _NC + lax.axis_index("c")
    sems_in = (si0, si1)
    sems_out = (so0, so1)
    i0 = wid * 512
    pltpu.sync_copy(idx_hbm.at[:, pl.ds(pl.multiple_of(i0, 512), 512)], idx_v)

    def start_gather(q, b):
        j = q // 4
        t = lax.rem(q, 4)
        pltpu.async_copy(
            table_hbm.at[idx_v.at[j, pl.ds(pl.multiple_of(128 * t, 128), 128)]],
            rows_v.at[b],
            sems_in[b],
        )

    def wait_gather(b):
        pltpu.make_async_copy(
            table_hbm.at[pl.ds(0, 128)], rows_v.at[b], sems_in[b]
        ).wait()

    def start_out(q, b):
        j = q // 4
        ti = 4 * wid + lax.rem(q, 4)
        pltpu.async_copy(
            tout_v.at[b], out_hbm.at[j, :, ti], sems_out[b]
        )

    def drain_out(b):
        pltpu.make_async_copy(
            tout_v.at[b], out_hbm.at[0, :, 0], sems_out[b]
        ).wait()

    start_gather(0, 0)

    def step(g, carry):
        for b in range(2):
            q = 2 * g + b
            wait_gather(b)

            @pl.when(q + 1 < _JC * 4)
            def _():
                start_gather(q + 1, 1 - b)

            @pl.when(q >= 2)
            def _():
                drain_out(b)

            _transpose_rows(rows_v.at[b], tout_v.at[b])
            start_out(q, b)

        return carry

    lax.fori_loop(0, _JC * 2, step, 0)
    drain_out(0)
    drain_out(1)


def kernel(x, table):
    rows, cols = x.shape
    xi = x.T.astype(jnp.int32)
    mesh = plsc.VectorSubcoreMesh(core_axis_name="c", subcore_axis_name="s")

    relayout = pl.kernel(
        _relayout_body,
        mesh=mesh,
        out_type=jax.ShapeDtypeStruct((_V // 4, 128), jnp.float32),
        scratch_types=[
            pltpu.VMEM((2, 32, 128), jnp.float32),
            pltpu.VMEM((2, 32, 128), jnp.float32),
            pltpu.SemaphoreType.DMA,
            pltpu.SemaphoreType.DMA,
            pltpu.SemaphoreType.DMA,
            pltpu.SemaphoreType.DMA,
        ],
        compiler_params=pltpu.CompilerParams(needs_layout_passes=False),
    )

    gather = pl.kernel(
        _gather_body,
        mesh=mesh,
        out_type=jax.ShapeDtypeStruct((_JC, 4, rows // 128, 8, 128), jnp.float32),
        scratch_types=[
            pltpu.VMEM((_JC, 512), jnp.int32),
            pltpu.VMEM((2, 128, D), jnp.float32),
            pltpu.VMEM((2, 4, 8, 128), jnp.float32),
            pltpu.SemaphoreType.DMA,
            pltpu.SemaphoreType.DMA,
            pltpu.SemaphoreType.DMA,
            pltpu.SemaphoreType.DMA,
        ],
        compiler_params=pltpu.CompilerParams(
            use_tc_tiling_on_sc=False, needs_layout_passes=False
        ),
    )

    tail = table[_TR_FULL * 128:].reshape(16, 128)
    table_lin = relayout(table.T, tail).reshape(_V, D)
    out5 = gather(table_lin, xi)
    return jnp.transpose(out5, (2, 4, 0, 1, 3)).reshape(rows, cols, D)
